# Initial kernel scaffold; baseline (speedup 1.0000x reference)
import jax
import jax.numpy as jnp
from jax import lax
from jax.experimental import pallas as pl
from jax.experimental.pallas import tpu as pltpu

B = 64
H = 1024
W = 1024
NPIX = H * W
HCHUNKS = 16
CHUNK = NPIX // HCHUNKS
SUB = 8192
ROWS = 128


def _hist_kernel(img_ref, hist_ref):
    c = pl.program_id(1)
    acc = jnp.zeros((16, 16), jnp.float32)
    v = img_ref[0]
    for s in range(CHUNK // SUB):
        vs = v[:, s * SUB:(s + 1) * SUB]
        hs = jnp.floor(vs * (1.0 / 16.0))
        ls = vs - hs * 16.0
        ih = lax.broadcasted_iota(jnp.float32, (16, SUB), 0)
        hi_oh = jnp.where(hs == ih, 1.0, 0.0)
        lo_oh = jnp.where(ls == ih, 1.0, 0.0)
        acc = acc + lax.dot_general(
            hi_oh, lo_oh, (((1,), (1,)), ((), ())),
            preferred_element_type=jnp.float32)

    @pl.when(c == 0)
    def _():
        hist_ref[0] = acc

    @pl.when(c != 0)
    def _():
        hist_ref[0] = hist_ref[0] + acc


def _main_kernel(img_ref, hist_ref, out_ref):
    hist = hist_ref[0]
    cdf = hist
    lane = lax.broadcasted_iota(jnp.int32, (1, 256), 1)
    for k in (1, 2, 4, 8, 16, 32, 64, 128):
        shifted = pltpu.roll(cdf, k, axis=1)
        cdf = cdf + jnp.where(lane >= k, shifted, 0.0)
    cdf_min = jnp.min(jnp.where(hist > 0.0, cdf, 3.4e38), axis=1,
                      keepdims=True)
    scale = 255.0 / jnp.maximum(jnp.float32(NPIX) - cdf_min, 1.0)
    lut = jnp.clip(jnp.round((cdf - cdf_min) * scale), 0.0, 255.0) * (
        1.0 / 255.0)
    lut_a = lut[:, :128]
    lut_b = lut[:, 128:]

    def equalize(x):
        r = x.shape[0]
        idx = jnp.round(x).astype(jnp.int32)
        idl = idx & 127
        ga = jnp.take_along_axis(jnp.broadcast_to(lut_a, (r, 128)), idl,
                                 axis=1)
        gb = jnp.take_along_axis(jnp.broadcast_to(lut_b, (r, 128)), idl,
                                 axis=1)
        return jnp.where(idx >= 128, gb, ga)

    zrow = jnp.zeros((1, W), jnp.float32)
    zcol = jnp.zeros((ROWS, 1), jnp.float32)
    acc_s = jnp.zeros((1, W), jnp.float32)
    acc_q = jnp.zeros((1, W), jnp.float32)
    for c in range(H // ROWS):
        r0 = c * ROWS
        lo = max(r0 - 1, 0)
        hi = min(r0 + ROWS + 1, H)
        eq = equalize(img_ref[0, lo:hi, :])
        if r0 == 0:
            eq = jnp.concatenate([zrow, eq], axis=0)
        if hi == H:
            eq = jnp.concatenate([eq, zrow], axis=0)
        up = eq[:-2, :]
        ctr = eq[1:-1, :]
        dn = eq[2:, :]
        lft = jnp.concatenate([ctr[:, 1:], zcol], axis=1)
        rgt = jnp.concatenate([zcol, ctr[:, :-1]], axis=1)
        lap = up + dn + lft + rgt - 4.0 * ctr
        acc_s = acc_s + jnp.sum(lap, axis=0, keepdims=True)
        acc_q = acc_q + jnp.sum(lap * lap, axis=0, keepdims=True)
    s = jnp.sum(acc_s, axis=1, keepdims=True)
    q = jnp.sum(acc_q, axis=1, keepdims=True)
    out_ref[0, 0:1, :] = jnp.broadcast_to(s, (1, 128))
    out_ref[0, 1:2, :] = jnp.broadcast_to(q, (1, 128))


def _hist_pass(img):
    imgc = img.reshape(B, HCHUNKS, CHUNK)
    return pl.pallas_call(
        _hist_kernel,
        grid=(B, HCHUNKS),
        in_specs=[pl.BlockSpec((1, 1, CHUNK), lambda i, c: (i, c, 0))],
        out_specs=pl.BlockSpec((1, 16, 16), lambda i, c: (i, 0, 0)),
        out_shape=jax.ShapeDtypeStruct((B, 16, 16), jnp.float32),
        compiler_params=pltpu.CompilerParams(
            dimension_semantics=("parallel", "arbitrary")),
    )(imgc)


def _main_pass(img, hist256):
    return pl.pallas_call(
        _main_kernel,
        grid=(B,),
        in_specs=[
            pl.BlockSpec((1, H, W), lambda i: (i, 0, 0)),
            pl.BlockSpec((1, 1, 256), lambda i: (i, 0, 0)),
        ],
        out_specs=pl.BlockSpec((1, 2, 128), lambda i: (i, 0, 0)),
        out_shape=jax.ShapeDtypeStruct((B, 2, 128), jnp.float32),
        compiler_params=pltpu.CompilerParams(
            dimension_semantics=("parallel",),
            vmem_limit_bytes=56 * 1024 * 1024),
    )(img, hist256)


def kernel(image, kernel):
    del kernel
    img = image.reshape(B, H, W)
    hist = _hist_pass(img)
    hist256 = hist.reshape(B, 1, 256)
    parts = _main_pass(img, hist256)
    sums = parts[:, 0, 0]
    sqs = parts[:, 1, 0]
    n = jnp.float32(B * NPIX)
    tot_s = jnp.sum(sums)
    tot_q = jnp.sum(sqs)
    return (tot_q - tot_s * tot_s / n) / (n - 1.0)


# speedup vs baseline: 396.6851x; 396.6851x over previous
import jax
import jax.numpy as jnp
from jax import lax
from jax.experimental import pallas as pl
from jax.experimental.pallas import tpu as pltpu

B = 64
H = 1024
W = 1024
NPIX = H * W
HCHUNKS = 16
CHUNK = NPIX // HCHUNKS
SUB = 8192
ROWS = 128


def _hist_kernel(img_ref, hist_ref):
    c = pl.program_id(1)
    acc = jnp.zeros((16, 16), jnp.float32)
    v = img_ref[0]
    for s in range(CHUNK // SUB):
        vs = v[:, s * SUB:(s + 1) * SUB]
        idx = jnp.round(vs).astype(jnp.int32)
        hs = idx >> 4
        ls = idx & 15
        ih = lax.broadcasted_iota(jnp.int32, (16, SUB), 0)
        hi_oh = jnp.where(hs == ih, 1.0, 0.0)
        lo_oh = jnp.where(ls == ih, 1.0, 0.0)
        acc = acc + lax.dot_general(
            hi_oh, lo_oh, (((1,), (1,)), ((), ())),
            preferred_element_type=jnp.float32)

    @pl.when(c == 0)
    def _():
        hist_ref[0] = acc

    @pl.when(c != 0)
    def _():
        hist_ref[0] = hist_ref[0] + acc


def _main_kernel(img_ref, hist_ref, out_ref):
    hist = hist_ref[0]
    cdf = hist
    lane = lax.broadcasted_iota(jnp.int32, (1, 256), 1)
    for k in (1, 2, 4, 8, 16, 32, 64, 128):
        shifted = pltpu.roll(cdf, k, axis=1)
        cdf = cdf + jnp.where(lane >= k, shifted, 0.0)
    cdf_min = jnp.min(jnp.where(hist > 0.0, cdf, 3.4e38), axis=1,
                      keepdims=True)
    scale = 255.0 / jnp.maximum(jnp.float32(NPIX) - cdf_min, 1.0)
    lut = jnp.clip(jnp.round((cdf - cdf_min) * scale), 0.0, 255.0) * (
        1.0 / 255.0)
    lut_a = lut[:, :128]
    lut_b = lut[:, 128:]

    def equalize(x):
        r = x.shape[0]
        idx = jnp.round(x).astype(jnp.int32)
        idl = idx & 127
        ga = jnp.take_along_axis(jnp.broadcast_to(lut_a, (r, 128)), idl,
                                 axis=1)
        gb = jnp.take_along_axis(jnp.broadcast_to(lut_b, (r, 128)), idl,
                                 axis=1)
        return jnp.where(idx >= 128, gb, ga)

    zrow = jnp.zeros((1, W), jnp.float32)
    zcol = jnp.zeros((ROWS, 1), jnp.float32)
    acc_s = jnp.zeros((1, W), jnp.float32)
    acc_q = jnp.zeros((1, W), jnp.float32)
    for c in range(H // ROWS):
        r0 = c * ROWS
        lo = max(r0 - 1, 0)
        hi = min(r0 + ROWS + 1, H)
        eq = equalize(img_ref[0, lo:hi, :])
        if r0 == 0:
            eq = jnp.concatenate([zrow, eq], axis=0)
        if hi == H:
            eq = jnp.concatenate([eq, zrow], axis=0)
        up = eq[:-2, :]
        ctr = eq[1:-1, :]
        dn = eq[2:, :]
        lft = jnp.concatenate([ctr[:, 1:], zcol], axis=1)
        rgt = jnp.concatenate([zcol, ctr[:, :-1]], axis=1)
        lap = up + dn + lft + rgt - 4.0 * ctr
        acc_s = acc_s + jnp.sum(lap, axis=0, keepdims=True)
        acc_q = acc_q + jnp.sum(lap * lap, axis=0, keepdims=True)
    s = jnp.sum(acc_s, axis=1, keepdims=True)
    q = jnp.sum(acc_q, axis=1, keepdims=True)
    out_ref[0, 0:1, :] = jnp.broadcast_to(s, (1, 128))
    out_ref[0, 1:2, :] = jnp.broadcast_to(q, (1, 128))


def _hist_pass(img):
    imgc = img.reshape(B * HCHUNKS, 1, CHUNK)
    return pl.pallas_call(
        _hist_kernel,
        grid=(B, HCHUNKS),
        in_specs=[pl.BlockSpec((1, 1, CHUNK),
                               lambda i, c: (i * HCHUNKS + c, 0, 0))],
        out_specs=pl.BlockSpec((1, 16, 16), lambda i, c: (i, 0, 0)),
        out_shape=jax.ShapeDtypeStruct((B, 16, 16), jnp.float32),
        compiler_params=pltpu.CompilerParams(
            dimension_semantics=("parallel", "arbitrary")),
    )(imgc)


def _main_pass(img, hist256):
    return pl.pallas_call(
        _main_kernel,
        grid=(B,),
        in_specs=[
            pl.BlockSpec((1, H, W), lambda i: (i, 0, 0)),
            pl.BlockSpec((1, 1, 256), lambda i: (i, 0, 0)),
        ],
        out_specs=pl.BlockSpec((1, 2, 128), lambda i: (i, 0, 0)),
        out_shape=jax.ShapeDtypeStruct((B, 2, 128), jnp.float32),
        compiler_params=pltpu.CompilerParams(
            dimension_semantics=("parallel",),
            vmem_limit_bytes=56 * 1024 * 1024),
    )(img, hist256)


def kernel(image, kernel):
    del kernel
    img = image.reshape(B, H, W)
    hist = _hist_pass(img)
    hist256 = hist.reshape(B, 1, 256)
    parts = _main_pass(img, hist256)
    sums = parts[:, 0, 0]
    sqs = parts[:, 1, 0]
    n = jnp.float32(B * NPIX)
    tot_s = jnp.sum(sums)
    tot_q = jnp.sum(sqs)
    return (tot_q - tot_s * tot_s / n) / (n - 1.0)


# speedup vs baseline: 448.9034x; 1.1316x over previous
import jax
import jax.numpy as jnp
from jax import lax
from jax.experimental import pallas as pl
from jax.experimental.pallas import tpu as pltpu

B = 64
H = 1024
W = 1024
NPIX = H * W
HCHUNKS = 16
CHUNK = NPIX // HCHUNKS
SUB = 8192
ROWS = 128
_PROBE = 0


def _hist_kernel(img_ref, hist_ref):
    acc = jnp.zeros((16, 16), jnp.float32)
    v = img_ref[0]
    ih = lax.broadcasted_iota(jnp.int32, (16, SUB), 0)
    for s in range(NPIX // SUB):
        vs = v[:, s * SUB:(s + 1) * SUB]
        idx = jnp.round(vs).astype(jnp.int32)
        hs = idx >> 4
        ls = idx & 15
        hi_oh = jnp.where(hs == ih, 1.0, 0.0)
        lo_oh = jnp.where(ls == ih, 1.0, 0.0)
        acc = acc + lax.dot_general(
            hi_oh, lo_oh, (((1,), (1,)), ((), ())),
            preferred_element_type=jnp.float32)
    hist_ref[0] = acc


def _main_kernel(img_ref, hist_ref, out_ref):
    hist = hist_ref[0]
    cdf = hist
    lane = lax.broadcasted_iota(jnp.int32, (1, 256), 1)
    for k in (1, 2, 4, 8, 16, 32, 64, 128):
        shifted = pltpu.roll(cdf, k, axis=1)
        cdf = cdf + jnp.where(lane >= k, shifted, 0.0)
    cdf_min = jnp.min(jnp.where(hist > 0.0, cdf, 3.4e38), axis=1,
                      keepdims=True)
    scale = 255.0 / jnp.maximum(jnp.float32(NPIX) - cdf_min, 1.0)
    lut = jnp.clip(jnp.round((cdf - cdf_min) * scale), 0.0, 255.0) * (
        1.0 / 255.0)
    lut_a = lut[:, :128]
    lut_b = lut[:, 128:]

    def equalize(x):
        r = x.shape[0]
        idx = jnp.round(x).astype(jnp.int32)
        idl = idx & 127
        ga = jnp.take_along_axis(jnp.broadcast_to(lut_a, (r, 128)), idl,
                                 axis=1)
        gb = jnp.take_along_axis(jnp.broadcast_to(lut_b, (r, 128)), idl,
                                 axis=1)
        return jnp.where(idx >= 128, gb, ga)

    zrow = jnp.zeros((1, W), jnp.float32)
    zcol = jnp.zeros((ROWS, 1), jnp.float32)
    acc_s = jnp.zeros((1, W), jnp.float32)
    acc_q = jnp.zeros((1, W), jnp.float32)
    for c in range(H // ROWS):
        r0 = c * ROWS
        lo = max(r0 - 1, 0)
        hi = min(r0 + ROWS + 1, H)
        eq = equalize(img_ref[0, lo:hi, :])
        if r0 == 0:
            eq = jnp.concatenate([zrow, eq], axis=0)
        if hi == H:
            eq = jnp.concatenate([eq, zrow], axis=0)
        up = eq[:-2, :]
        ctr = eq[1:-1, :]
        dn = eq[2:, :]
        lft = jnp.concatenate([ctr[:, 1:], zcol], axis=1)
        rgt = jnp.concatenate([zcol, ctr[:, :-1]], axis=1)
        lap = up + dn + lft + rgt - 4.0 * ctr
        acc_s = acc_s + jnp.sum(lap, axis=0, keepdims=True)
        acc_q = acc_q + jnp.sum(lap * lap, axis=0, keepdims=True)
    s = jnp.sum(acc_s, axis=1, keepdims=True)
    q = jnp.sum(acc_q, axis=1, keepdims=True)
    out_ref[0, 0:1, :] = jnp.broadcast_to(s, (1, 128))
    out_ref[0, 1:2, :] = jnp.broadcast_to(q, (1, 128))


def _hist_pass(img):
    imgc = img.reshape(B, 1, NPIX)
    return pl.pallas_call(
        _hist_kernel,
        grid=(B,),
        in_specs=[pl.BlockSpec((1, 1, NPIX), lambda i: (i, 0, 0))],
        out_specs=pl.BlockSpec((1, 16, 16), lambda i: (i, 0, 0)),
        out_shape=jax.ShapeDtypeStruct((B, 16, 16), jnp.float32),
        compiler_params=pltpu.CompilerParams(
            dimension_semantics=("arbitrary",),
            vmem_limit_bytes=56 * 1024 * 1024),
    )(imgc)


def _main_pass(img, hist256):
    return pl.pallas_call(
        _main_kernel,
        grid=(B,),
        in_specs=[
            pl.BlockSpec((1, H, W), lambda i: (i, 0, 0)),
            pl.BlockSpec((1, 1, 256), lambda i: (i, 0, 0)),
        ],
        out_specs=pl.BlockSpec((1, 2, 128), lambda i: (i, 0, 0)),
        out_shape=jax.ShapeDtypeStruct((B, 2, 128), jnp.float32),
        compiler_params=pltpu.CompilerParams(
            dimension_semantics=("arbitrary",),
            vmem_limit_bytes=56 * 1024 * 1024),
    )(img, hist256)


def kernel(image, kernel):
    del kernel
    img = image.reshape(B, H, W)
    hist = _hist_pass(img)
    if _PROBE == 1:
        return jnp.sum(hist)
    if _PROBE == 2:
        parts = _main_pass(img, jnp.zeros((B, 1, 256), jnp.float32) + 1.0)
        return jnp.sum(parts)
    hist256 = hist.reshape(B, 1, 256)
    parts = _main_pass(img, hist256)
    sums = parts[:, 0, 0]
    sqs = parts[:, 1, 0]
    n = jnp.float32(B * NPIX)
    tot_s = jnp.sum(sums)
    tot_q = jnp.sum(sqs)
    return (tot_q - tot_s * tot_s / n) / (n - 1.0)


# speedup vs baseline: 467.0803x; 1.0405x over previous
import jax
import jax.numpy as jnp
from jax import lax
from jax.experimental import pallas as pl
from jax.experimental.pallas import tpu as pltpu

B = 64
H = 1024
W = 1024
NPIX = H * W
HCHUNKS = 16
CHUNK = NPIX // HCHUNKS
SUB = 8192
ROWS = 32
_PROBE = 0


def _hist_kernel(img_ref, hist_ref):
    acc = jnp.zeros((16, 16), jnp.float32)
    v = img_ref[0]
    ih = lax.broadcasted_iota(jnp.int32, (16, SUB), 0)
    for s in range(NPIX // SUB):
        vs = v[:, s * SUB:(s + 1) * SUB]
        idx = jnp.round(vs).astype(jnp.int32)
        hs = idx >> 4
        ls = idx & 15
        hi_oh = jnp.where(hs == ih, 1.0, 0.0)
        lo_oh = jnp.where(ls == ih, 1.0, 0.0)
        acc = acc + lax.dot_general(
            hi_oh, lo_oh, (((1,), (1,)), ((), ())),
            preferred_element_type=jnp.float32)
    hist_ref[0] = acc


def _main_kernel(img_ref, hist_ref, out_ref, eq_ref):
    hist = hist_ref[0]
    cdf = hist
    lane = lax.broadcasted_iota(jnp.int32, (1, 256), 1)
    for k in (1, 2, 4, 8, 16, 32, 64, 128):
        shifted = pltpu.roll(cdf, k, axis=1)
        cdf = cdf + jnp.where(lane >= k, shifted, 0.0)
    cdf_min = jnp.min(jnp.where(hist > 0.0, cdf, 3.4e38), axis=1,
                      keepdims=True)
    scale = 255.0 / jnp.maximum(jnp.float32(NPIX) - cdf_min, 1.0)
    lut = jnp.clip(jnp.round((cdf - cdf_min) * scale), 0.0, 255.0) * (
        1.0 / 255.0)
    lut_a = jnp.broadcast_to(lut[:, :128], (8, 128))
    lut_b = jnp.broadcast_to(lut[:, 128:], (8, 128))

    eq_ref[0:8, :] = jnp.zeros((8, W), jnp.float32)
    eq_ref[H + 8:H + 16, :] = jnp.zeros((8, W), jnp.float32)
    for g in range(H // 8):
        x = img_ref[0, 8 * g:8 * g + 8, :]
        idx = jnp.round(x).astype(jnp.int32)
        idl = idx & 127
        ga = jnp.take_along_axis(lut_a, idl, axis=1)
        gb = jnp.take_along_axis(lut_b, idl, axis=1)
        eq_ref[8 + 8 * g:16 + 8 * g, :] = jnp.where(idx >= 128, gb, ga)

    zcol = jnp.zeros((ROWS, 1), jnp.float32)
    acc_s = jnp.zeros((8, W), jnp.float32)
    acc_q = jnp.zeros((8, W), jnp.float32)
    for c in range(H // ROWS):
        r0 = c * ROWS
        up = eq_ref[7 + r0:7 + r0 + ROWS, :]
        ctr = eq_ref[8 + r0:8 + r0 + ROWS, :]
        dn = eq_ref[9 + r0:9 + r0 + ROWS, :]
        lft = jnp.concatenate([ctr[:, 1:], zcol], axis=1)
        rgt = jnp.concatenate([zcol, ctr[:, :-1]], axis=1)
        lap = up + dn + lft + rgt - 4.0 * ctr
        sq = lap * lap
        for b in range(ROWS // 8):
            acc_s = acc_s + lap[8 * b:8 * b + 8, :]
            acc_q = acc_q + sq[8 * b:8 * b + 8, :]
    s = jnp.sum(acc_s, keepdims=True)
    q = jnp.sum(acc_q, keepdims=True)
    out_ref[0, 0:1, :] = jnp.broadcast_to(s, (1, 128))
    out_ref[0, 1:2, :] = jnp.broadcast_to(q, (1, 128))


def _hist_pass(img):
    imgc = img.reshape(B, 1, NPIX)
    return pl.pallas_call(
        _hist_kernel,
        grid=(B,),
        in_specs=[pl.BlockSpec((1, 1, NPIX), lambda i: (i, 0, 0))],
        out_specs=pl.BlockSpec((1, 16, 16), lambda i: (i, 0, 0)),
        out_shape=jax.ShapeDtypeStruct((B, 16, 16), jnp.float32),
        compiler_params=pltpu.CompilerParams(
            dimension_semantics=("arbitrary",),
            vmem_limit_bytes=56 * 1024 * 1024),
    )(imgc)


def _main_pass(img, hist256):
    return pl.pallas_call(
        _main_kernel,
        grid=(B,),
        in_specs=[
            pl.BlockSpec((1, H, W), lambda i: (i, 0, 0)),
            pl.BlockSpec((1, 1, 256), lambda i: (i, 0, 0)),
        ],
        out_specs=pl.BlockSpec((1, 2, 128), lambda i: (i, 0, 0)),
        out_shape=jax.ShapeDtypeStruct((B, 2, 128), jnp.float32),
        scratch_shapes=[pltpu.VMEM((H + 16, W), jnp.float32)],
        compiler_params=pltpu.CompilerParams(
            dimension_semantics=("arbitrary",),
            vmem_limit_bytes=56 * 1024 * 1024),
    )(img, hist256)


def kernel(image, kernel):
    del kernel
    img = image.reshape(B, H, W)
    hist = _hist_pass(img)
    if _PROBE == 1:
        return jnp.sum(hist)
    if _PROBE == 2:
        parts = _main_pass(img, jnp.zeros((B, 1, 256), jnp.float32) + 1.0)
        return jnp.sum(parts)
    hist256 = hist.reshape(B, 1, 256)
    parts = _main_pass(img, hist256)
    sums = parts[:, 0, 0]
    sqs = parts[:, 1, 0]
    n = jnp.float32(B * NPIX)
    tot_s = jnp.sum(sums)
    tot_q = jnp.sum(sqs)
    return (tot_q - tot_s * tot_s / n) / (n - 1.0)


# speedup vs baseline: 468.1103x; 1.0022x over previous
import jax
import jax.numpy as jnp
from jax import lax
from jax.experimental import pallas as pl
from jax.experimental.pallas import tpu as pltpu

B = 64
H = 1024
W = 1024
NPIX = H * W
HCHUNKS = 16
CHUNK = NPIX // HCHUNKS
SUB = 8192
ROWS = 16
_PROBE = 0


def _hist_kernel(img_ref, hist_ref):
    acc = jnp.zeros((16, 16), jnp.float32)
    v = img_ref[0]
    ih = lax.broadcasted_iota(jnp.int32, (16, SUB), 0)
    for s in range(NPIX // SUB):
        vs = v[:, s * SUB:(s + 1) * SUB]
        idx = jnp.round(vs).astype(jnp.int32)
        hs = idx >> 4
        ls = idx & 15
        hi_oh = jnp.where(hs == ih, 1.0, 0.0)
        lo_oh = jnp.where(ls == ih, 1.0, 0.0)
        acc = acc + lax.dot_general(
            hi_oh, lo_oh, (((1,), (1,)), ((), ())),
            preferred_element_type=jnp.float32)
    hist_ref[0] = acc


def _main_kernel(img_ref, hist_ref, out_ref, eq_ref):
    hist = hist_ref[0]
    cdf = hist
    lane = lax.broadcasted_iota(jnp.int32, (1, 256), 1)
    for k in (1, 2, 4, 8, 16, 32, 64, 128):
        shifted = pltpu.roll(cdf, k, axis=1)
        cdf = cdf + jnp.where(lane >= k, shifted, 0.0)
    cdf_min = jnp.min(jnp.where(hist > 0.0, cdf, 3.4e38), axis=1,
                      keepdims=True)
    scale = 255.0 / jnp.maximum(jnp.float32(NPIX) - cdf_min, 1.0)
    lut = jnp.clip(jnp.round((cdf - cdf_min) * scale), 0.0, 255.0) * (
        1.0 / 255.0)
    lut_a = jnp.broadcast_to(lut[:, :128], (8, 128))
    lut_b = jnp.broadcast_to(lut[:, 128:], (8, 128))

    eq_ref[0:8, :] = jnp.zeros((8, W), jnp.float32)
    eq_ref[H + 8:H + 16, :] = jnp.zeros((8, W), jnp.float32)
    for g in range(H // 8):
        x = img_ref[0, 8 * g:8 * g + 8, :]
        idx = jnp.round(x).astype(jnp.int32)
        idl = idx & 127
        ga = jnp.take_along_axis(lut_a, idl, axis=1)
        gb = jnp.take_along_axis(lut_b, idl, axis=1)
        eq_ref[8 + 8 * g:16 + 8 * g, :] = jnp.where(idx >= 128, gb, ga)

    zcol = jnp.zeros((ROWS, 1), jnp.float32)
    acc_s = jnp.zeros((8, W), jnp.float32)
    acc_q = jnp.zeros((8, W), jnp.float32)
    for c in range(H // ROWS):
        r0 = c * ROWS
        w = eq_ref[7 + r0:9 + r0 + ROWS, :]
        up = w[:ROWS, :]
        ctr = w[1:ROWS + 1, :]
        dn = w[2:, :]
        lft = jnp.concatenate([ctr[:, 1:], zcol], axis=1)
        rgt = jnp.concatenate([zcol, ctr[:, :-1]], axis=1)
        lap = up + dn + lft + rgt - 4.0 * ctr
        sq = lap * lap
        for b in range(ROWS // 8):
            acc_s = acc_s + lap[8 * b:8 * b + 8, :]
            acc_q = acc_q + sq[8 * b:8 * b + 8, :]
    s = jnp.sum(acc_s, keepdims=True)
    q = jnp.sum(acc_q, keepdims=True)
    out_ref[0, 0:1, :] = jnp.broadcast_to(s, (1, 128))
    out_ref[0, 1:2, :] = jnp.broadcast_to(q, (1, 128))


def _hist_pass(img):
    imgc = img.reshape(B, 1, NPIX)
    return pl.pallas_call(
        _hist_kernel,
        grid=(B,),
        in_specs=[pl.BlockSpec((1, 1, NPIX), lambda i: (i, 0, 0))],
        out_specs=pl.BlockSpec((1, 16, 16), lambda i: (i, 0, 0)),
        out_shape=jax.ShapeDtypeStruct((B, 16, 16), jnp.float32),
        compiler_params=pltpu.CompilerParams(
            dimension_semantics=("arbitrary",),
            vmem_limit_bytes=56 * 1024 * 1024),
    )(imgc)


def _main_pass(img, hist256):
    return pl.pallas_call(
        _main_kernel,
        grid=(B,),
        in_specs=[
            pl.BlockSpec((1, H, W), lambda i: (i, 0, 0)),
            pl.BlockSpec((1, 1, 256), lambda i: (i, 0, 0)),
        ],
        out_specs=pl.BlockSpec((1, 2, 128), lambda i: (i, 0, 0)),
        out_shape=jax.ShapeDtypeStruct((B, 2, 128), jnp.float32),
        scratch_shapes=[pltpu.VMEM((H + 16, W), jnp.float32)],
        compiler_params=pltpu.CompilerParams(
            dimension_semantics=("arbitrary",),
            vmem_limit_bytes=56 * 1024 * 1024),
    )(img, hist256)


def kernel(image, kernel):
    del kernel
    img = image.reshape(B, H, W)
    hist = _hist_pass(img)
    if _PROBE == 1:
        return jnp.sum(hist)
    if _PROBE == 2:
        parts = _main_pass(img, jnp.zeros((B, 1, 256), jnp.float32) + 1.0)
        return jnp.sum(parts)
    hist256 = hist.reshape(B, 1, 256)
    parts = _main_pass(img, hist256)
    sums = parts[:, 0, 0]
    sqs = parts[:, 1, 0]
    n = jnp.float32(B * NPIX)
    tot_s = jnp.sum(sums)
    tot_q = jnp.sum(sqs)
    return (tot_q - tot_s * tot_s / n) / (n - 1.0)
